# Initial kernel scaffold; baseline (speedup 1.0000x reference)
import jax
import jax.numpy as jnp
from jax.experimental import pallas as pl
from jax.experimental.pallas import tpu as pltpu

N = 8192
M = 8192
TR = 512
GRID = N // TR


def _chamfer_kernel(t_ref, st_ref, out_ref, cmin_ref, s1_ref):
    i = pl.program_id(0)

    t = t_ref[...]
    sx = st_ref[0:1, :]
    sy = st_ref[1:2, :]
    sz = st_ref[2:3, :]

    dx = t[:, 0:1] - sx
    dy = t[:, 1:2] - sy
    dz = t[:, 2:3] - sz
    d = dx * dx + dy * dy + dz * dz

    row_min = jnp.min(d, axis=1)
    s1_part = jnp.sum(jnp.sqrt(jnp.maximum(row_min, 0.0)))
    col_min = jnp.min(d, axis=0, keepdims=True)

    @pl.when(i == 0)
    def _init():
        cmin_ref[...] = col_min
        s1_ref[0, 0] = s1_part

    @pl.when(i > 0)
    def _acc():
        cmin_ref[...] = jnp.minimum(cmin_ref[...], col_min)
        s1_ref[0, 0] = s1_ref[0, 0] + s1_part

    @pl.when(i == GRID - 1)
    def _finish():
        dist2 = jnp.maximum(cmin_ref[...], 0.0)
        s2 = jnp.sum(jnp.sqrt(dist2))
        s1 = s1_ref[0, 0]
        out_ref[0, 0] = (s1 / N + s2 / M) * 5.0


@jax.jit
def _chamfer(state_x, target):
    st = state_x.T
    loss = pl.pallas_call(
        _chamfer_kernel,
        grid=(GRID,),
        in_specs=[
            pl.BlockSpec((TR, 3), lambda i: (i, 0)),
            pl.BlockSpec((3, M), lambda i: (0, 0)),
        ],
        out_specs=pl.BlockSpec(memory_space=pltpu.SMEM),
        out_shape=jax.ShapeDtypeStruct((1, 1), jnp.float32),
        scratch_shapes=[
            pltpu.VMEM((1, M), jnp.float32),
            pltpu.SMEM((1, 1), jnp.float32),
        ],
    )(target, st)
    return loss[0, 0]


def kernel(state_x, target):
    return _chamfer(state_x, target)


# speedup vs baseline: 1.4679x; 1.4679x over previous
import jax
import jax.numpy as jnp
from jax.experimental import pallas as pl
from jax.experimental.pallas import tpu as pltpu

N = 8192
M = 8192
TR = 512
GRID = N // TR


def _chamfer_kernel(t_ref, st2_ref, b2_ref, out_ref, cmin_ref, s1_ref):
    i = pl.program_id(0)

    t = t_ref[...]
    a2 = t[:, 0:1] * t[:, 0:1] + t[:, 1:2] * t[:, 1:2] + t[:, 2:3] * t[:, 2:3]

    ab = jnp.dot(t.astype(jnp.bfloat16), st2_ref[...],
                 preferred_element_type=jnp.float32)
    d = (a2 + b2_ref[...]) + ab

    row_min = jnp.min(d, axis=1)
    s1_part = jnp.sum(jnp.sqrt(jnp.maximum(row_min, 0.0)))
    col_min = jnp.min(d, axis=0, keepdims=True)

    @pl.when(i == 0)
    def _init():
        cmin_ref[...] = col_min
        s1_ref[0, 0] = s1_part

    @pl.when(i > 0)
    def _acc():
        cmin_ref[...] = jnp.minimum(cmin_ref[...], col_min)
        s1_ref[0, 0] = s1_ref[0, 0] + s1_part

    @pl.when(i == GRID - 1)
    def _finish():
        dist2 = jnp.maximum(cmin_ref[...], 0.0)
        s2 = jnp.sum(jnp.sqrt(dist2))
        s1 = s1_ref[0, 0]
        out_ref[0, 0] = (s1 / N + s2 / M) * 5.0


@jax.jit
def _chamfer(state_x, target):
    st2 = (-2.0 * state_x).astype(jnp.bfloat16).T
    b2 = jnp.sum(state_x * state_x, axis=1)[None, :]
    loss = pl.pallas_call(
        _chamfer_kernel,
        grid=(GRID,),
        in_specs=[
            pl.BlockSpec((TR, 3), lambda i: (i, 0)),
            pl.BlockSpec((3, M), lambda i: (0, 0)),
            pl.BlockSpec((1, M), lambda i: (0, 0)),
        ],
        out_specs=pl.BlockSpec(memory_space=pltpu.SMEM),
        out_shape=jax.ShapeDtypeStruct((1, 1), jnp.float32),
        scratch_shapes=[
            pltpu.VMEM((1, M), jnp.float32),
            pltpu.SMEM((1, 1), jnp.float32),
        ],
    )(target, st2, b2)
    return loss[0, 0]


def kernel(state_x, target):
    return _chamfer(state_x, target)


# speedup vs baseline: 1.6495x; 1.1237x over previous
import jax
import jax.numpy as jnp
from jax.experimental import pallas as pl
from jax.experimental.pallas import tpu as pltpu

N = 8192
M = 8192
TR = 512
GRID = N // TR


def _chamfer_kernel(a_ref, b_ref, out_ref, cmin_ref, s1_ref):
    i = pl.program_id(0)

    d = jnp.dot(a_ref[...], b_ref[...],
                preferred_element_type=jnp.float32)

    row_min = jnp.min(d, axis=1)
    s1_part = jnp.sum(jnp.sqrt(jnp.maximum(row_min, 0.0)))
    col_min = jnp.min(d, axis=0, keepdims=True)

    @pl.when(i == 0)
    def _init():
        cmin_ref[...] = col_min
        s1_ref[0, 0] = s1_part

    @pl.when(i > 0)
    def _acc():
        cmin_ref[...] = jnp.minimum(cmin_ref[...], col_min)
        s1_ref[0, 0] = s1_ref[0, 0] + s1_part

    @pl.when(i == GRID - 1)
    def _finish():
        dist2 = jnp.maximum(cmin_ref[...], 0.0)
        s2 = jnp.sum(jnp.sqrt(dist2))
        s1 = s1_ref[0, 0]
        out_ref[0, 0] = (s1 / N + s2 / M) * 5.0


def _hi_lo(x):
    xi = jax.lax.bitcast_convert_type(x, jnp.uint32)
    hi_f = jax.lax.bitcast_convert_type(xi & jnp.uint32(0xFFFF0000), jnp.float32)
    hi = hi_f.astype(jnp.bfloat16)
    lo = (x - hi_f).astype(jnp.bfloat16)
    return hi, lo


@jax.jit
def _chamfer(state_x, target):
    a2 = jnp.sum(target * target, axis=1)
    b2 = jnp.sum(state_x * state_x, axis=1)
    a2h, a2l = _hi_lo(a2)
    b2h, b2l = _hi_lo(b2)
    one = jnp.ones((N,), jnp.bfloat16)
    zero = jnp.zeros((N,), jnp.bfloat16)
    A = jnp.stack(
        [target[:, 0].astype(jnp.bfloat16),
         target[:, 1].astype(jnp.bfloat16),
         target[:, 2].astype(jnp.bfloat16),
         a2h, a2l, one, one, zero], axis=1)
    B = jnp.stack(
        [(-2.0 * state_x[:, 0]).astype(jnp.bfloat16),
         (-2.0 * state_x[:, 1]).astype(jnp.bfloat16),
         (-2.0 * state_x[:, 2]).astype(jnp.bfloat16),
         one, one, b2h, b2l, zero], axis=0)

    loss = pl.pallas_call(
        _chamfer_kernel,
        grid=(GRID,),
        in_specs=[
            pl.BlockSpec((TR, 8), lambda i: (i, 0)),
            pl.BlockSpec((8, M), lambda i: (0, 0)),
        ],
        out_specs=pl.BlockSpec(memory_space=pltpu.SMEM),
        out_shape=jax.ShapeDtypeStruct((1, 1), jnp.float32),
        scratch_shapes=[
            pltpu.VMEM((1, M), jnp.float32),
            pltpu.SMEM((1, 1), jnp.float32),
        ],
    )(A, B)
    return loss[0, 0]


def kernel(state_x, target):
    return _chamfer(state_x, target)


# speedup vs baseline: 1.7436x; 1.0570x over previous
import jax
import jax.numpy as jnp
from jax.experimental import pallas as pl
from jax.experimental.pallas import tpu as pltpu

N = 8192
M = 8192
TR = 1024
GRID = N // TR


CC = 2048
NCC = M // CC


def _chamfer_kernel(a_ref, b_ref, out_ref, cmin_ref, s1_ref):
    i = pl.program_id(0)

    a = a_ref[...]
    row_mins = []
    col_mins = []
    for c in range(NCC):
        dc = jnp.dot(a, b_ref[:, c * CC:(c + 1) * CC],
                     preferred_element_type=jnp.float32)
        row_mins.append(jnp.min(dc, axis=1))
        col_mins.append(jnp.min(dc, axis=0))
    row_min = jnp.minimum(jnp.minimum(row_mins[0], row_mins[1]),
                          jnp.minimum(row_mins[2], row_mins[3]))
    s1_part = jnp.sum(jnp.sqrt(jnp.maximum(row_min, 0.0)))
    col_min = jnp.concatenate(col_mins)[None, :]

    @pl.when(i == 0)
    def _init():
        cmin_ref[...] = col_min
        s1_ref[0, 0] = s1_part

    @pl.when(i > 0)
    def _acc():
        cmin_ref[...] = jnp.minimum(cmin_ref[...], col_min)
        s1_ref[0, 0] = s1_ref[0, 0] + s1_part

    @pl.when(i == GRID - 1)
    def _finish():
        dist2 = jnp.maximum(cmin_ref[...], 0.0)
        s2 = jnp.sum(jnp.sqrt(dist2))
        s1 = s1_ref[0, 0]
        out_ref[0, 0] = (s1 / N + s2 / M) * 5.0


def _hi_lo(x):
    xi = jax.lax.bitcast_convert_type(x, jnp.uint32)
    hi_f = jax.lax.bitcast_convert_type(xi & jnp.uint32(0xFFFF0000), jnp.float32)
    hi = hi_f.astype(jnp.bfloat16)
    lo = (x - hi_f).astype(jnp.bfloat16)
    return hi, lo


@jax.jit
def _chamfer(state_x, target):
    a2 = jnp.sum(target * target, axis=1)
    b2 = jnp.sum(state_x * state_x, axis=1)
    a2h, a2l = _hi_lo(a2)
    b2h, b2l = _hi_lo(b2)
    one = jnp.ones((N,), jnp.bfloat16)
    zero = jnp.zeros((N,), jnp.bfloat16)
    A = jnp.stack(
        [target[:, 0].astype(jnp.bfloat16),
         target[:, 1].astype(jnp.bfloat16),
         target[:, 2].astype(jnp.bfloat16),
         a2h, a2l, one, one, zero], axis=1)
    B = jnp.stack(
        [(-2.0 * state_x[:, 0]).astype(jnp.bfloat16),
         (-2.0 * state_x[:, 1]).astype(jnp.bfloat16),
         (-2.0 * state_x[:, 2]).astype(jnp.bfloat16),
         one, one, b2h, b2l, zero], axis=0)

    loss = pl.pallas_call(
        _chamfer_kernel,
        grid=(GRID,),
        in_specs=[
            pl.BlockSpec((TR, 8), lambda i: (i, 0)),
            pl.BlockSpec((8, M), lambda i: (0, 0)),
        ],
        out_specs=pl.BlockSpec(memory_space=pltpu.SMEM),
        out_shape=jax.ShapeDtypeStruct((1, 1), jnp.float32),
        scratch_shapes=[
            pltpu.VMEM((1, M), jnp.float32),
            pltpu.SMEM((1, 1), jnp.float32),
        ],
    )(A, B)
    return loss[0, 0]


def kernel(state_x, target):
    return _chamfer(state_x, target)
